# baseline (speedup 1.0000x reference)
import functools

import jax
import jax.numpy as jnp
from jax import lax
from jax.experimental import pallas as pl
from jax.experimental.pallas import tpu as pltpu
from jax.experimental.pallas import tpu_sc as plsc

_N = 10000
_E = 160000
_D = 256
_P = 512
_BLK = 1000
_H = 128
_K = 80
_CHUNKS = _E // (16 * _K)
_NP = 10240
_STRIPE = _NP // 16
_DHALF = _CHUNKS // 2 + 1


def _make_agg(with_deg):
    def body(*args):
        if with_deg:
            (xlo, xhi, pk3, zrows, zdeg, ones_in,
             out_lo, out_hi, deg0_out, deg1_out,
             acc, dega, packed, sb0, db0, sb1, db1, sb2, db2, sb3, db3,
             rows0, rows1, rows2, ones_buf,
             gsem0, gsem1, gsem2, ssem0, ssem1, ssem2) = args
        else:
            (xlo, xhi, pk3, zrows,
             out_lo, out_hi,
             acc, packed, sb0, db0, sb1, db1, sb2, db2, sb3, db3,
             rows0, rows1, rows2,
             gsem0, gsem1, gsem2, ssem0, ssem1, ssem2) = args
        c = lax.axis_index("c")
        s = lax.axis_index("s")
        stripe = s * _STRIPE

        pltpu.sync_copy(zrows.at[pl.ds(stripe, _STRIPE)],
                        acc.at[pl.ds(stripe, _STRIPE)])
        if with_deg:
            pltpu.sync_copy(zdeg.at[pl.ds(stripe, _STRIPE)],
                            dega.at[pl.ds(stripe, _STRIPE)])
            pltpu.sync_copy(ones_in, ones_buf)
        pltpu.sync_copy(pk3.at[s], packed)
        plsc.subcore_barrier()

        def unpack(j, sbuf, dbuf):
            base = j * _K
            for i in range(_K // 16):
                v = packed[pl.ds(base + i * 16, 16)]
                sbuf[pl.ds(i * 16, 16)] = v & 16383
                dbuf[pl.ds(i * 16, 16)] = lax.shift_right_logical(v, 14)

        def deg_for(j, dbuf):
            if not with_deg:
                return
            cond = jnp.where(c == 0, j < _DHALF, j >= _DHALF)

            @pl.when(cond)
            def _():
                pltpu.sync_copy(ones_buf, dega.at[dbuf], add=True)

        def run(xc):
            rows = [rows0, rows1, rows2]
            sbs = [sb0, sb1, sb2, sb3]
            dbs = [db0, db1, db2, db3]
            gsems = [gsem0, gsem1, gsem2]
            ssems = [ssem0, ssem1, ssem2]

            unpack(0, sbs[0], dbs[0])
            pltpu.async_copy(xc.at[sbs[0]], rows[0], gsems[0])
            unpack(1, sbs[1], dbs[1])
            pltpu.async_copy(xc.at[sbs[1]], rows[1], gsems[1])

            def step(j, a3, a4):
                b3 = (a3 + 2) % 3
                b4 = (a4 + 2) % 4
                pltpu.make_async_copy(xc.at[sbs[a4]], rows[a3],
                                      gsems[a3]).wait()

                @pl.when(j > 0)
                def _():
                    pltpu.make_async_copy(rows[b3], acc.at[dbs[(a4 + 3) % 4]],
                                          ssems[b3]).wait()

                unpack(j + 2, sbs[b4], dbs[b4])
                pltpu.async_copy(xc.at[sbs[b4]], rows[b3], gsems[b3])
                pltpu.async_copy(rows[a3], acc.at[dbs[a4]], ssems[a3],
                                 add=True)
                deg_for(j, dbs[a4])

            def body_p(p, carry):
                j0 = 12 * p
                for q in range(12):
                    step(j0 + q, q % 3, q % 4)
                return carry

            lax.fori_loop(0, _CHUNKS // 12, body_p, 0)
            for q in range(3):
                step(120 + q, q, q)
            pltpu.make_async_copy(xc.at[sbs[3]], rows[0], gsems[0]).wait()
            pltpu.make_async_copy(rows[2], acc.at[dbs[2]], ssems[2]).wait()
            pltpu.async_copy(rows[0], acc.at[dbs[3]], ssems[0], add=True)
            deg_for(_CHUNKS - 2, dbs[3])
            pltpu.make_async_copy(xc.at[sbs[0]], rows[1], gsems[1]).wait()
            pltpu.make_async_copy(rows[0], acc.at[dbs[3]], ssems[0]).wait()
            pltpu.sync_copy(rows[1], acc.at[dbs[0]], add=True)
            deg_for(_CHUNKS - 1, dbs[0])

        @pl.when(c == 0)
        def _():
            run(xlo)

        @pl.when(c == 1)
        def _():
            run(xhi)

        plsc.subcore_barrier()

        @pl.when(c == 0)
        def _():
            pltpu.sync_copy(acc.at[pl.ds(stripe, _STRIPE)],
                            out_lo.at[pl.ds(stripe, _STRIPE)])
            if with_deg:
                pltpu.sync_copy(dega.at[pl.ds(stripe, _STRIPE)],
                                deg0_out.at[pl.ds(stripe, _STRIPE)])

        @pl.when(c == 1)
        def _():
            pltpu.sync_copy(acc.at[pl.ds(stripe, _STRIPE)],
                            out_hi.at[pl.ds(stripe, _STRIPE)])
            if with_deg:
                pltpu.sync_copy(dega.at[pl.ds(stripe, _STRIPE)],
                                deg1_out.at[pl.ds(stripe, _STRIPE)])

    outs = [
        jax.ShapeDtypeStruct((_NP, _H), jnp.float32),
        jax.ShapeDtypeStruct((_NP, _H), jnp.float32),
    ]
    scratch = [
        pltpu.VMEM_SHARED((_NP, _H), jnp.float32),
    ]
    if with_deg:
        outs += [
            jax.ShapeDtypeStruct((_NP,), jnp.float32),
            jax.ShapeDtypeStruct((_NP,), jnp.float32),
        ]
        scratch += [pltpu.VMEM_SHARED((_NP,), jnp.float32)]
    scratch += [
        pltpu.VMEM((_CHUNKS * _K,), jnp.int32),
        pltpu.VMEM((_K,), jnp.int32),
        pltpu.VMEM((_K,), jnp.int32),
        pltpu.VMEM((_K,), jnp.int32),
        pltpu.VMEM((_K,), jnp.int32),
        pltpu.VMEM((_K,), jnp.int32),
        pltpu.VMEM((_K,), jnp.int32),
        pltpu.VMEM((_K,), jnp.int32),
        pltpu.VMEM((_K,), jnp.int32),
        pltpu.VMEM((_K, _H), jnp.float32),
        pltpu.VMEM((_K, _H), jnp.float32),
        pltpu.VMEM((_K, _H), jnp.float32),
    ]
    if with_deg:
        scratch += [pltpu.VMEM((_K,), jnp.float32)]
    scratch += [pltpu.SemaphoreType.DMA] * 6

    return pl.kernel(
        body,
        out_type=outs,
        mesh=plsc.VectorSubcoreMesh(core_axis_name="c", subcore_axis_name="s"),
        scratch_types=scratch,
    )


def _bdot(a, b):
    return jnp.dot(a.astype(jnp.bfloat16), b.astype(jnp.bfloat16),
                   preferred_element_type=jnp.float32)


def _dense_xr_body(x, w, b, out):
    out[...] = _bdot(x[...], w[...]) + b[...]


def _dense_xr(x, W, b):
    nb = _N // _BLK
    return pl.pallas_call(
        _dense_xr_body,
        grid=(nb,),
        in_specs=[
            pl.BlockSpec((_BLK, _D), lambda i: (i, 0)),
            pl.BlockSpec((_D, _D), lambda i: (0, 0)),
            pl.BlockSpec((1, _D), lambda i: (0, 0)),
        ],
        out_specs=pl.BlockSpec((_BLK, _D), lambda i: (i, 0)),
        out_shape=jax.ShapeDtypeStruct((_N, _D), jnp.float32),
    )(x, W, b)


def _dense_hr_body(hlo, hhi, w, b, out):
    wr = w[...]
    out[...] = (_bdot(hlo[...], wr[0:_H, :]) + _bdot(hhi[...], wr[_H:_D, :])
                + b[...])


def _dense_hr(hlo, hhi, W, b):
    nb = _N // _BLK
    return pl.pallas_call(
        _dense_hr_body,
        grid=(nb,),
        in_specs=[
            pl.BlockSpec((_BLK, _H), lambda i: (i, 0)),
            pl.BlockSpec((_BLK, _H), lambda i: (i, 0)),
            pl.BlockSpec((_D, _D), lambda i: (0, 0)),
            pl.BlockSpec((1, _D), lambda i: (0, 0)),
        ],
        out_specs=pl.BlockSpec((_BLK, _D), lambda i: (i, 0)),
        out_shape=jax.ShapeDtypeStruct((_N, _D), jnp.float32),
    )(hlo, hhi, W, b)


def _dense1_body(s1lo, s1hi, r1, deg0, deg1, w1l, olo, ohi):
    inv = 1.0 / jnp.maximum(deg0[...] + deg1[...], 1.0)
    wl = w1l[...]
    h = _bdot(s1lo[...] * inv, wl[0:_H, :])
    h = h + _bdot(s1hi[...] * inv, wl[_H:_D, :])
    h = jnp.maximum(h + r1[...], 0.0)
    olo[...] = h[:, 0:_H]
    ohi[...] = h[:, _H:_D]


def _dense1(s_lo, s_hi, r1, deg0, deg1, W1l):
    nb = _N // _BLK
    return pl.pallas_call(
        _dense1_body,
        grid=(nb,),
        in_specs=[
            pl.BlockSpec((_BLK, _H), lambda i: (i, 0)),
            pl.BlockSpec((_BLK, _H), lambda i: (i, 0)),
            pl.BlockSpec((_BLK, _D), lambda i: (i, 0)),
            pl.BlockSpec((_BLK, 1), lambda i: (i, 0)),
            pl.BlockSpec((_BLK, 1), lambda i: (i, 0)),
            pl.BlockSpec((_D, _D), lambda i: (0, 0)),
        ],
        out_specs=[
            pl.BlockSpec((_BLK, _H), lambda i: (i, 0)),
            pl.BlockSpec((_BLK, _H), lambda i: (i, 0)),
        ],
        out_shape=[
            jax.ShapeDtypeStruct((_N, _H), jnp.float32),
            jax.ShapeDtypeStruct((_N, _H), jnp.float32),
        ],
    )(s_lo, s_hi, r1, deg0, deg1, W1l)


def _dense2_body(s2lo, s2hi, r2, deg0, deg1, w2l, wh, bh, a, la, out):
    inv = 1.0 / jnp.maximum(deg0[...] + deg1[...], 1.0)
    wl = w2l[...]
    h2 = _bdot(s2lo[...] * inv, wl[0:_H, :])
    h2 = h2 + _bdot(s2hi[...] * inv, wl[_H:_D, :])
    h2 = h2 + r2[...]
    z = _bdot(jnp.maximum(h2, 0.0), wh[...]) + bh[...]
    zs = lax.dot_general(z.astype(jnp.bfloat16), a[...].astype(jnp.bfloat16),
                         (((1,), (1,)), ((), ())),
                         preferred_element_type=jnp.float32)
    alpha = 1.0 / (1.0 + jnp.exp(-la[...]))
    out[...] = alpha * zs + (1.0 - alpha) * z


def _dense2(s_lo, s_hi, r2, deg0, deg1, W2l, Wh, bh, A_norm, la):
    nb = _N // _BLK
    return pl.pallas_call(
        _dense2_body,
        grid=(nb,),
        in_specs=[
            pl.BlockSpec((_BLK, _H), lambda i: (i, 0)),
            pl.BlockSpec((_BLK, _H), lambda i: (i, 0)),
            pl.BlockSpec((_BLK, _D), lambda i: (i, 0)),
            pl.BlockSpec((_BLK, 1), lambda i: (i, 0)),
            pl.BlockSpec((_BLK, 1), lambda i: (i, 0)),
            pl.BlockSpec((_D, _D), lambda i: (0, 0)),
            pl.BlockSpec((_D, _P), lambda i: (0, 0)),
            pl.BlockSpec((1, _P), lambda i: (0, 0)),
            pl.BlockSpec((_P, _P), lambda i: (0, 0)),
            pl.BlockSpec((1, 1), lambda i: (0, 0)),
        ],
        out_specs=pl.BlockSpec((_BLK, _P), lambda i: (i, 0)),
        out_shape=jax.ShapeDtypeStruct((_N, _P), jnp.float32),
    )(s_lo, s_hi, r2, deg0, deg1, W2l, Wh, bh, A_norm, la)


def kernel(x, edge_index, W1l, W1r, b1, W2l, W2r, b2, Wh, bh, logit_alpha,
           A_norm):
    packed = (edge_index[0] | (edge_index[1] << 14)).reshape(16, _CHUNKS * _K)
    xlo = x[:, :_H]
    xhi = x[:, _H:]
    zrows = jnp.zeros((_NP, _H), jnp.float32)
    zdeg = jnp.zeros((_NP,), jnp.float32)
    ones_in = jnp.ones((_K,), jnp.float32)

    s1lo, s1hi, d0, d1 = _make_agg(True)(
        xlo, xhi, packed, zrows, zdeg, ones_in)
    r1 = _dense_xr(x, W1r, b1.reshape(1, _D))
    d0 = d0.reshape(_NP, 1)
    d1 = d1.reshape(_NP, 1)
    h1lo, h1hi = _dense1(s1lo, s1hi, r1, d0, d1, W1l)
    s2lo, s2hi = _make_agg(False)(h1lo, h1hi, packed, zrows)
    r2 = _dense_hr(h1lo, h1hi, W2r, b2.reshape(1, _D))
    out = _dense2(s2lo, s2hi, r2, d0, d1, W2l, Wh, bh.reshape(1, _P),
                  A_norm, logit_alpha.reshape(1, 1).astype(jnp.float32))
    return out


# speedup vs baseline: 1.0058x; 1.0058x over previous
import functools

import jax
import jax.numpy as jnp
from jax import lax
from jax.experimental import pallas as pl
from jax.experimental.pallas import tpu as pltpu
from jax.experimental.pallas import tpu_sc as plsc

_N = 10000
_E = 160000
_D = 256
_P = 512
_BLK = 1000
_H = 128
_K = 80
_CHUNKS = _E // (16 * _K)
_NP = 10240
_STRIPE = _NP // 16
_DHALF = _CHUNKS // 2 + 1


def _make_agg(with_deg):
    def body(*args):
        if with_deg:
            (xlo, xhi, pk3, zrows, zdeg, ones_in,
             out_lo, out_hi, deg0_out, deg1_out,
             acc, dega, packed, sb0, db0, sb1, db1, sb2, db2, sb3, db3,
             rows0, rows1, rows2, ones_buf,
             gsem0, gsem1, gsem2, ssem0, ssem1, ssem2) = args
        else:
            (xlo, xhi, pk3, zrows,
             out_lo, out_hi,
             acc, packed, sb0, db0, sb1, db1, sb2, db2, sb3, db3,
             rows0, rows1, rows2,
             gsem0, gsem1, gsem2, ssem0, ssem1, ssem2) = args
        c = lax.axis_index("c")
        s = lax.axis_index("s")
        stripe = s * _STRIPE

        pltpu.sync_copy(zrows.at[pl.ds(stripe, _STRIPE)],
                        acc.at[pl.ds(stripe, _STRIPE)])
        if with_deg:
            pltpu.sync_copy(zdeg.at[pl.ds(stripe, _STRIPE)],
                            dega.at[pl.ds(stripe, _STRIPE)])
            pltpu.sync_copy(ones_in, ones_buf)
        pltpu.sync_copy(pk3.at[s], packed)
        plsc.subcore_barrier()

        def unpack(j, sbuf, dbuf):
            base = j * _K
            for i in range(_K // 16):
                v = packed[pl.ds(base + i * 16, 16)]
                sbuf[pl.ds(i * 16, 16)] = v & 16383
                dbuf[pl.ds(i * 16, 16)] = lax.shift_right_logical(v, 14)

        def deg_for(j, dbuf):
            if not with_deg:
                return
            cond = jnp.where(c == 0, j < _DHALF, j >= _DHALF)

            @pl.when(cond)
            def _():
                pltpu.sync_copy(ones_buf, dega.at[dbuf], add=True)

        def run(xc):
            rows = [rows0, rows1, rows2]
            sbs = [sb0, sb1, sb2, sb3]
            dbs = [db0, db1, db2, db3]
            gsems = [gsem0, gsem1, gsem2]
            ssems = [ssem0, ssem1, ssem2]

            unpack(0, sbs[0], dbs[0])
            pltpu.async_copy(xc.at[sbs[0]], rows[0], gsems[0])
            unpack(1, sbs[1], dbs[1])
            pltpu.async_copy(xc.at[sbs[1]], rows[1], gsems[1])

            def step(j, a3, a4):
                b3 = (a3 + 2) % 3
                b4 = (a4 + 2) % 4
                pltpu.make_async_copy(xc.at[sbs[a4]], rows[a3],
                                      gsems[a3]).wait()

                @pl.when(j > 0)
                def _():
                    pltpu.make_async_copy(rows[b3], acc.at[dbs[(a4 + 3) % 4]],
                                          ssems[b3]).wait()

                unpack(j + 2, sbs[b4], dbs[b4])
                pltpu.async_copy(xc.at[sbs[b4]], rows[b3], gsems[b3])
                pltpu.async_copy(rows[a3], acc.at[dbs[a4]], ssems[a3],
                                 add=True)
                deg_for(j, dbs[a4])

            def body_p(p, carry):
                j0 = 12 * p
                for q in range(12):
                    step(j0 + q, q % 3, q % 4)
                return carry

            lax.fori_loop(0, _CHUNKS // 12, body_p, 0)
            for q in range(3):
                step(120 + q, q, q)
            pltpu.make_async_copy(xc.at[sbs[3]], rows[0], gsems[0]).wait()
            pltpu.make_async_copy(rows[2], acc.at[dbs[2]], ssems[2]).wait()
            pltpu.async_copy(rows[0], acc.at[dbs[3]], ssems[0], add=True)
            deg_for(_CHUNKS - 2, dbs[3])
            pltpu.make_async_copy(xc.at[sbs[0]], rows[1], gsems[1]).wait()
            pltpu.make_async_copy(rows[0], acc.at[dbs[3]], ssems[0]).wait()
            pltpu.sync_copy(rows[1], acc.at[dbs[0]], add=True)
            deg_for(_CHUNKS - 1, dbs[0])

        @pl.when(c == 0)
        def _():
            run(xlo)

        @pl.when(c == 1)
        def _():
            run(xhi)

        plsc.subcore_barrier()

        @pl.when(c == 0)
        def _():
            pltpu.sync_copy(acc.at[pl.ds(stripe, _STRIPE)],
                            out_lo.at[pl.ds(stripe, _STRIPE)])
            if with_deg:
                pltpu.sync_copy(dega.at[pl.ds(stripe, _STRIPE)],
                                deg0_out.at[pl.ds(stripe, _STRIPE)])

        @pl.when(c == 1)
        def _():
            pltpu.sync_copy(acc.at[pl.ds(stripe, _STRIPE)],
                            out_hi.at[pl.ds(stripe, _STRIPE)])
            if with_deg:
                pltpu.sync_copy(dega.at[pl.ds(stripe, _STRIPE)],
                                deg1_out.at[pl.ds(stripe, _STRIPE)])

    outs = [
        jax.ShapeDtypeStruct((_NP, _H), jnp.float32),
        jax.ShapeDtypeStruct((_NP, _H), jnp.float32),
    ]
    scratch = [
        pltpu.VMEM_SHARED((_NP, _H), jnp.float32),
    ]
    if with_deg:
        outs += [
            jax.ShapeDtypeStruct((_NP,), jnp.float32),
            jax.ShapeDtypeStruct((_NP,), jnp.float32),
        ]
        scratch += [pltpu.VMEM_SHARED((_NP,), jnp.float32)]
    scratch += [
        pltpu.VMEM((_CHUNKS * _K,), jnp.int32),
        pltpu.VMEM((_K,), jnp.int32),
        pltpu.VMEM((_K,), jnp.int32),
        pltpu.VMEM((_K,), jnp.int32),
        pltpu.VMEM((_K,), jnp.int32),
        pltpu.VMEM((_K,), jnp.int32),
        pltpu.VMEM((_K,), jnp.int32),
        pltpu.VMEM((_K,), jnp.int32),
        pltpu.VMEM((_K,), jnp.int32),
        pltpu.VMEM((_K, _H), jnp.float32),
        pltpu.VMEM((_K, _H), jnp.float32),
        pltpu.VMEM((_K, _H), jnp.float32),
    ]
    if with_deg:
        scratch += [pltpu.VMEM((_K,), jnp.float32)]
    scratch += [pltpu.SemaphoreType.DMA] * 6

    return pl.kernel(
        body,
        out_type=outs,
        mesh=plsc.VectorSubcoreMesh(core_axis_name="c", subcore_axis_name="s"),
        scratch_types=scratch,
    )


def _bdot(a, b):
    return jnp.dot(a.astype(jnp.bfloat16), b.astype(jnp.bfloat16),
                   preferred_element_type=jnp.float32)


def _dense1_body(s1lo, s1hi, x, deg0, deg1, w1l, w1r, b1, olo, ohi):
    inv = 1.0 / jnp.maximum(deg0[...] + deg1[...], 1.0)
    wl = w1l[...]
    h = jnp.dot(s1lo[...] * inv, wl[0:_H, :],
                preferred_element_type=jnp.float32)
    h = h + jnp.dot(s1hi[...] * inv, wl[_H:_D, :],
                    preferred_element_type=jnp.float32)
    h = h + jnp.dot(x[...], w1r[...], preferred_element_type=jnp.float32)
    h = jnp.maximum(h + b1[...], 0.0)
    olo[...] = h[:, 0:_H]
    ohi[...] = h[:, _H:_D]


def _dense1(s_lo, s_hi, x, deg0, deg1, W1l, W1r, b1):
    nb = _N // _BLK
    return pl.pallas_call(
        _dense1_body,
        grid=(nb,),
        in_specs=[
            pl.BlockSpec((_BLK, _H), lambda i: (i, 0)),
            pl.BlockSpec((_BLK, _H), lambda i: (i, 0)),
            pl.BlockSpec((_BLK, _D), lambda i: (i, 0)),
            pl.BlockSpec((_BLK, 1), lambda i: (i, 0)),
            pl.BlockSpec((_BLK, 1), lambda i: (i, 0)),
            pl.BlockSpec((_D, _D), lambda i: (0, 0)),
            pl.BlockSpec((_D, _D), lambda i: (0, 0)),
            pl.BlockSpec((1, _D), lambda i: (0, 0)),
        ],
        out_specs=[
            pl.BlockSpec((_BLK, _H), lambda i: (i, 0)),
            pl.BlockSpec((_BLK, _H), lambda i: (i, 0)),
        ],
        out_shape=[
            jax.ShapeDtypeStruct((_N, _H), jnp.float32),
            jax.ShapeDtypeStruct((_N, _H), jnp.float32),
        ],
    )(s_lo, s_hi, x, deg0, deg1, W1l, W1r, b1)


def _dense2_body(s2lo, s2hi, h1lo, h1hi, deg0, deg1, w2l, w2r, b2, wh, bh,
                 a, la, out):
    inv = 1.0 / jnp.maximum(deg0[...] + deg1[...], 1.0)
    wl = w2l[...]
    wr = w2r[...]
    h2 = jnp.dot(s2lo[...] * inv, wl[0:_H, :],
                 preferred_element_type=jnp.float32)
    h2 = h2 + jnp.dot(s2hi[...] * inv, wl[_H:_D, :],
                      preferred_element_type=jnp.float32)
    h2 = h2 + jnp.dot(h1lo[...], wr[0:_H, :],
                      preferred_element_type=jnp.float32)
    h2 = h2 + jnp.dot(h1hi[...], wr[_H:_D, :],
                      preferred_element_type=jnp.float32)
    h2 = h2 + b2[...]
    z = jnp.dot(jnp.maximum(h2, 0.0), wh[...],
                preferred_element_type=jnp.float32) + bh[...]
    zs = lax.dot_general(z, a[...], (((1,), (1,)), ((), ())),
                         preferred_element_type=jnp.float32)
    alpha = 1.0 / (1.0 + jnp.exp(-la[...]))
    out[...] = alpha * zs + (1.0 - alpha) * z


def _dense2(s_lo, s_hi, h1lo, h1hi, deg0, deg1, W2l, W2r, b2, Wh, bh,
            A_norm, la):
    nb = _N // _BLK
    return pl.pallas_call(
        _dense2_body,
        grid=(nb,),
        in_specs=[
            pl.BlockSpec((_BLK, _H), lambda i: (i, 0)),
            pl.BlockSpec((_BLK, _H), lambda i: (i, 0)),
            pl.BlockSpec((_BLK, _H), lambda i: (i, 0)),
            pl.BlockSpec((_BLK, _H), lambda i: (i, 0)),
            pl.BlockSpec((_BLK, 1), lambda i: (i, 0)),
            pl.BlockSpec((_BLK, 1), lambda i: (i, 0)),
            pl.BlockSpec((_D, _D), lambda i: (0, 0)),
            pl.BlockSpec((_D, _D), lambda i: (0, 0)),
            pl.BlockSpec((1, _D), lambda i: (0, 0)),
            pl.BlockSpec((_D, _P), lambda i: (0, 0)),
            pl.BlockSpec((1, _P), lambda i: (0, 0)),
            pl.BlockSpec((_P, _P), lambda i: (0, 0)),
            pl.BlockSpec((1, 1), lambda i: (0, 0)),
        ],
        out_specs=pl.BlockSpec((_BLK, _P), lambda i: (i, 0)),
        out_shape=jax.ShapeDtypeStruct((_N, _P), jnp.float32),
    )(s_lo, s_hi, h1lo, h1hi, deg0, deg1, W2l, W2r, b2, Wh, bh, A_norm, la)


def kernel(x, edge_index, W1l, W1r, b1, W2l, W2r, b2, Wh, bh, logit_alpha,
           A_norm):
    packed = (edge_index[0] | (edge_index[1] << 14)).reshape(16, _CHUNKS * _K)
    xlo = x[:, :_H]
    xhi = x[:, _H:]
    zrows = jnp.zeros((_NP, _H), jnp.float32)
    zdeg = jnp.zeros((_NP,), jnp.float32)
    ones_in = jnp.ones((_K,), jnp.float32)

    s1lo, s1hi, d0, d1 = _make_agg(True)(
        xlo, xhi, packed, zrows, zdeg, ones_in)
    d0 = d0.reshape(_NP, 1)
    d1 = d1.reshape(_NP, 1)
    h1lo, h1hi = _dense1(s1lo, s1hi, x, d0, d1, W1l, W1r, b1.reshape(1, _D))
    s2lo, s2hi = _make_agg(False)(h1lo, h1hi, packed, zrows)
    out = _dense2(s2lo, s2hi, h1lo, h1hi, d0, d1, W2l, W2r,
                  b2.reshape(1, _D), Wh, bh.reshape(1, _P), A_norm,
                  logit_alpha.reshape(1, 1).astype(jnp.float32))
    return out
